# baseline (speedup 1.0000x reference)
import jax
import jax.numpy as jnp
from jax.experimental import pallas as pl
from jax.experimental.pallas import tpu as pltpu

NUM_TIMESTEPS = 1000
BETA_START = 0.0001
BETA_END = 0.02

_CH = 32


def _body(a_ref, c_ref, x_ref, n_ref, o_ref):
    a = a_ref[...].reshape(_CH, 1, 1, 1)
    c = c_ref[...].reshape(_CH, 1, 1, 1)
    o_ref[...] = a * x_ref[...] + c * n_ref[...]


def _tables():
    betas = jnp.linspace(BETA_START, BETA_END, NUM_TIMESTEPS, dtype=jnp.float32)
    alphas_cumprod = jnp.cumprod(1.0 - betas, axis=0)
    sac = jnp.sqrt(alphas_cumprod)
    somac = jnp.sqrt(1.0 - alphas_cumprod)
    return sac, somac


def kernel(x_start, t, noise):
    B, C, H, W = x_start.shape
    sac, somac = _tables()
    t32 = t.astype(jnp.int32)
    a = jnp.take(sac, t32, axis=0).reshape(B, 1)
    c = jnp.take(somac, t32, axis=0).reshape(B, 1)

    return pl.pallas_call(
        _body,
        grid=(B // _CH,),
        in_specs=[
            pl.BlockSpec((_CH, 1), lambda i: (i, 0)),
            pl.BlockSpec((_CH, 1), lambda i: (i, 0)),
            pl.BlockSpec((_CH, C, H, W), lambda i: (i, 0, 0, 0)),
            pl.BlockSpec((_CH, C, H, W), lambda i: (i, 0, 0, 0)),
        ],
        out_specs=pl.BlockSpec((_CH, C, H, W), lambda i: (i, 0, 0, 0)),
        out_shape=jax.ShapeDtypeStruct((B, C, H, W), jnp.float32),
    )(a, c, x_start, noise)


# speedup vs baseline: 1.7122x; 1.7122x over previous
import functools

import jax
import jax.numpy as jnp
from jax import lax
from jax.experimental import pallas as pl
from jax.experimental.pallas import tpu as pltpu
from jax.experimental.pallas import tpu_sc as plsc

NUM_TIMESTEPS = 1000
BETA_START = 0.0001
BETA_END = 0.02

_NC = 2
_NS = 16
_NW = _NC * _NS
_B = 1024
_F = 16384
_BPW = _B // _NW
_TPAD = 1024
_LANES = 16


def _tables():
    betas = jnp.linspace(BETA_START, BETA_END, NUM_TIMESTEPS, dtype=jnp.float32)
    alphas_cumprod = jnp.cumprod(1.0 - betas, axis=0)
    sac = jnp.sqrt(alphas_cumprod)
    somac = jnp.sqrt(1.0 - alphas_cumprod)
    pad = _TPAD - NUM_TIMESTEPS
    return jnp.pad(sac, (0, pad)), jnp.pad(somac, (0, pad))


@functools.partial(
    pl.kernel,
    out_type=jax.ShapeDtypeStruct((_B, _F), jnp.float32),
    mesh=plsc.VectorSubcoreMesh(
        core_axis_name="c", subcore_axis_name="s",
        num_cores=_NC, num_subcores=_NS),
    compiler_params=pltpu.CompilerParams(needs_layout_passes=False),
    scratch_types=[
        pltpu.VMEM((_BPW,), jnp.int32),
        pltpu.VMEM((_TPAD,), jnp.float32),
        pltpu.VMEM((_TPAD,), jnp.float32),
        pltpu.VMEM((_BPW + 16,), jnp.float32),
        pltpu.VMEM((_BPW + 16,), jnp.float32),
        pltpu.VMEM((_F,), jnp.float32),
        pltpu.VMEM((_F,), jnp.float32),
        pltpu.VMEM((_F,), jnp.float32),
        pltpu.VMEM((_F,), jnp.float32),
        pltpu.VMEM((_F,), jnp.float32),
        pltpu.VMEM((_F,), jnp.float32),
        pltpu.SemaphoreType.DMA,
        pltpu.SemaphoreType.DMA,
        pltpu.SemaphoreType.DMA,
        pltpu.SemaphoreType.DMA,
        pltpu.SemaphoreType.DMA,
        pltpu.SemaphoreType.DMA,
    ],
)
def _sc_q_sample(t_hbm, sac_hbm, somac_hbm, x_hbm, n_hbm, o_hbm,
                 t_v, sac_v, somac_v, a_v, c_v,
                 xb0, xb1, nb0, nb1, ob0, ob1,
                 xs0, xs1, ns0, ns1, os0, os1):
    wid = lax.axis_index("s") * _NC + lax.axis_index("c")
    base = wid * _BPW

    xbufs = (xb0, xb1)
    nbufs = (nb0, nb1)
    obufs = (ob0, ob1)
    xsems = (xs0, xs1)
    nsems = (ns0, ns1)
    osems = (os0, os1)

    def in_copies(r, s):
        cx = pltpu.make_async_copy(x_hbm.at[base + r], xbufs[s], xsems[s])
        cn = pltpu.make_async_copy(n_hbm.at[base + r], nbufs[s], nsems[s])
        return cx, cn

    def out_copy(r, s):
        return pltpu.make_async_copy(obufs[s], o_hbm.at[base + r], osems[s])

    pltpu.sync_copy(t_hbm.at[pl.ds(base, _BPW)], t_v)
    pltpu.sync_copy(sac_hbm, sac_v)
    pltpu.sync_copy(somac_hbm, somac_v)

    for s in (0, 1):
        cx, cn = in_copies(s, s)
        cx.start()
        cn.start()

    for j in range(_BPW // _LANES):
        idx = t_v[pl.ds(j * _LANES, _LANES)]
        a_v[pl.ds(16 + j * _LANES, _LANES)] = plsc.load_gather(sac_v, [idx])
        c_v[pl.ds(16 + j * _LANES, _LANES)] = plsc.load_gather(somac_v, [idx])

    def row_compute(r, s):
        rvec = jnp.full((_LANES,), 16, jnp.int32) + r
        a16 = plsc.load_gather(a_v, [rvec])
        c16 = plsc.load_gather(c_v, [rvec])
        xb, nb, ob = xbufs[s], nbufs[s], obufs[s]

        def body(k, carry):
            for u in range(8):
                off = (k * 8 + u) * _LANES
                sl = pl.ds(off, _LANES)
                ob[sl] = a16 * xb[sl] + c16 * nb[sl]
            return carry

        lax.fori_loop(0, _F // (8 * _LANES), body, 0)

    def do_row(r, s, first, last):
        cx, cn = in_copies(r, s)
        cx.wait()
        cn.wait()
        if not first:
            out_copy(r - 2, s).wait()
        row_compute(r, s)
        out_copy(r, s).start()
        if not last:
            cx, cn = in_copies(r + 2, s)
            cx.start()
            cn.start()

    tz = wid * 0
    do_row(tz + 0, 0, True, False)
    do_row(tz + 1, 1, True, False)

    def pair(i, carry):
        r = 2 + i * 2
        do_row(r, 0, False, False)
        do_row(r + 1, 1, False, False)
        return carry

    lax.fori_loop(0, (_BPW - 4) // 2, pair, 0)

    do_row(tz + _BPW - 2, 0, False, True)
    do_row(tz + _BPW - 1, 1, False, True)

    out_copy(_BPW - 2, 0).wait()
    out_copy(_BPW - 1, 1).wait()


def kernel(x_start, t, noise):
    B = x_start.shape[0]
    x = x_start.reshape(B, _F)
    n = noise.reshape(B, _F)
    sac, somac = _tables()
    t32 = t.astype(jnp.int32)
    out = _sc_q_sample(t32, sac, somac, x, n)
    return out.reshape(x_start.shape)
